# baseline (speedup 1.0000x reference)
import functools

import jax
import jax.numpy as jnp
from jax import lax
from jax.experimental import pallas as pl
from jax.experimental.pallas import tpu as pltpu
from jax.experimental.pallas import tpu_sc as plsc

F = 32
L = 16
CH = 128
SLAB = 8
NSLAB = F // SLAB
NPAIR = SLAB // 2


def _packed_slabs(emb, n_main):
    flats = []
    nt = n_main // CH
    for t in range(NSLAB):
        a = lax.slice(emb, (0, SLAB * t), (n_main, SLAB * (t + 1)))
        b = a.astype(jnp.bfloat16).reshape(nt, CH, NPAIR, 2)
        c = lax.bitcast_convert_type(b.transpose(0, 2, 1, 3), jnp.int32)
        flats.append(c.reshape(-1))
    return flats


def _packed_rem(emb, n_main, n):
    r = lax.slice(emb, (n_main, 0), (n, F))
    return lax.bitcast_convert_type(
        r.astype(jnp.bfloat16).reshape(n - n_main, F // 2, 2), jnp.int32)


def kernel(user_ids, item_ids, user_emb, item_emb, user_bias, item_bias, global_bias):
    B = user_ids.shape[0]
    N = user_emb.shape[0]
    info = plsc.get_sparse_core_info()
    NC, NS = info.num_cores, info.num_subcores
    NW = NC * NS
    b_per_w = B // NW
    n_ch = b_per_w // CH
    n_main = (N // CH) * CH
    n_rem = N - n_main
    n_rows = NPAIR * n_ch

    uid2d = jnp.asarray(user_ids, jnp.int32).reshape(B // CH, CH)
    iid2d = jnp.asarray(item_ids, jnp.int32).reshape(B // CH, CH)
    gb16 = jnp.broadcast_to(jnp.asarray(global_bias, jnp.float32).reshape(1), (L,))
    uflats = _packed_slabs(user_emb, n_main)
    iflats = _packed_slabs(item_emb, n_main)
    rem_u = _packed_rem(user_emb, n_main, N)
    rem_i = _packed_rem(item_emb, n_main, N)

    mesh = plsc.VectorSubcoreMesh(core_axis_name="c", subcore_axis_name="s")

    @functools.partial(
        pl.kernel,
        mesh=mesh,
        compiler_params=pltpu.CompilerParams(
            needs_layout_passes=False, use_tc_tiling_on_sc=False),
        out_type=jax.ShapeDtypeStruct((B,), jnp.float32),
        scratch_types=[
            pltpu.VMEM((n_ch, CH), jnp.int32),
            pltpu.VMEM((n_ch, CH), jnp.int32),
            pltpu.VMEM((n_rows, CH), jnp.int32),
            pltpu.VMEM((n_rows, CH), jnp.int32),
            pltpu.VMEM((NSLAB, n_rows, CH), jnp.int32),
            pltpu.VMEM((NSLAB, n_rows, CH), jnp.int32),
            pltpu.VMEM((n_ch, CH), jnp.float32),
            pltpu.VMEM((n_ch, CH), jnp.float32),
            pltpu.VMEM((n_rem, F // 2), jnp.int32),
            pltpu.VMEM((n_rem, F // 2), jnp.int32),
            pltpu.VMEM((b_per_w,), jnp.float32),
            pltpu.VMEM((L,), jnp.float32),
            pltpu.SemaphoreType.DMA,
        ],
    )
    def _k(uid_hbm, iid_hbm,
           uf0, uf1, uf2, uf3, if0, if1, if2, if3,
           remu_hbm, remi_hbm, ubias_hbm, ibias_hbm, gb_hbm,
           out_hbm,
           uidx, iidx, uoffs, ioffs, ubuf, vbuf, bu, bi,
           remu, remi, outb, gbv, sem):
        ufs = (uf0, uf1, uf2, uf3)
        ifs = (if0, if1, if2, if3)
        wid = lax.axis_index("s") * NC + lax.axis_index("c")
        row0 = wid * n_ch
        base = wid * b_per_w

        pltpu.sync_copy(uid_hbm.at[pl.ds(row0, n_ch)], uidx)
        pltpu.sync_copy(iid_hbm.at[pl.ds(row0, n_ch)], iidx)
        pltpu.sync_copy(gb_hbm, gbv)
        pltpu.sync_copy(remu_hbm, remu)
        pltpu.sync_copy(remi_hbm, remi)

        def offs_body(m, _):
            def sub_body(s, _):
                cols = pl.ds(s * L, L)
                for ids_ref, offs_ref in ((uidx, uoffs), (iidx, ioffs)):
                    r = ids_ref[m, cols]
                    rc = jnp.minimum(r, n_main - 1)
                    b0 = ((rc >> 7) << 9) + (rc & 127)
                    for p in range(NPAIR):
                        offs_ref[p * n_ch + m, cols] = b0 + p * CH
                return 0
            lax.fori_loop(0, CH // L, sub_body, 0)
            return 0

        lax.fori_loop(0, n_ch, offs_body, 0)

        copies = []
        for t in range(NSLAB):
            for j in range(n_rows):
                copies.append(pltpu.async_copy(
                    ufs[t].at[uoffs.at[j]], ubuf.at[t, j], sem))
                copies.append(pltpu.async_copy(
                    ifs[t].at[ioffs.at[j]], vbuf.at[t, j], sem))
        for m in range(n_ch):
            copies.append(pltpu.async_copy(
                ubias_hbm.at[uidx.at[m]], bu.at[m], sem))
            copies.append(pltpu.async_copy(
                ibias_hbm.at[iidx.at[m]], bi.at[m], sem))
        for c in copies:
            c.wait()

        gv = gbv[...]

        def fix_body(c16, _):
            m = c16 // (CH // L)
            cols = pl.ds((c16 % (CH // L)) * L, L)
            for ids_ref, buf, rem in ((uidx, ubuf, remu), (iidx, vbuf, remi)):
                r = ids_ref[m, cols]
                rmax = lax.reduce_max(r, axes=(0,))

                @pl.when(rmax >= n_main)
                def _():
                    tail = r >= n_main
                    rr = jnp.maximum(r - n_main, 0)
                    for t in range(NSLAB):
                        for p in range(NPAIR):
                            j = p * n_ch + m
                            cur = buf[t, j, cols]
                            g = plsc.load_gather(
                                rem, [rr, jnp.full((L,), NPAIR * t + p, jnp.int32)])
                            buf[t, j, cols] = jnp.where(tail, g, cur)
                return 0
            return 0

        lax.fori_loop(0, b_per_w // L, fix_body, 0)

        def acc_body(c16, _):
            m = c16 // (CH // L)
            cols = pl.ds((c16 % (CH // L)) * L, L)
            acc = gv + bu[m, cols] + bi[m, cols]
            for t in range(NSLAB):
                for p in range(NPAIR):
                    j = p * n_ch + m
                    ub = plsc.bitcast(ubuf[t, j, cols], jnp.bfloat16)
                    vb = plsc.bitcast(vbuf[t, j, cols], jnp.bfloat16)
                    ue, uo = plsc.unpack(ub, format=plsc.PackFormat.INTERLEAVED)
                    ve, vo = plsc.unpack(vb, format=plsc.PackFormat.INTERLEAVED)
                    acc = acc + ue * ve + uo * vo
            outb[pl.ds(c16 * L, L)] = acc
            return 0

        lax.fori_loop(0, b_per_w // L, acc_body, 0)

        pltpu.sync_copy(outb, out_hbm.at[pl.ds(base, b_per_w)])

    return _k(uid2d, iid2d, *uflats, *iflats, rem_u, rem_i,
              user_bias, item_bias, gb16)


# speedup vs baseline: 3.8970x; 3.8970x over previous
import functools

import jax
import jax.numpy as jnp
from jax import lax
from jax.experimental import pallas as pl
from jax.experimental.pallas import tpu as pltpu
from jax.experimental.pallas import tpu_sc as plsc

F = 32
L = 16
CH = 128
SLAB = 8
NSLAB = F // SLAB

_params = pltpu.CompilerParams(
    needs_layout_passes=False, use_tc_tiling_on_sc=False)


def _slab_flats(emb, n_main):
    flats = []
    nt = n_main // CH
    for t in range(NSLAB):
        a = lax.slice(emb, (0, SLAB * t), (n_main, SLAB * (t + 1)))
        flats.append(a.reshape(nt, CH, SLAB).transpose(0, 2, 1).reshape(-1))
    return flats


def kernel(user_ids, item_ids, user_emb, item_emb, user_bias, item_bias, global_bias):
    B = user_ids.shape[0]
    N = user_emb.shape[0]
    info = plsc.get_sparse_core_info()
    NC, NS = info.num_cores, info.num_subcores
    NW = NC * NS
    b_per_w = B // NW
    n_ch = b_per_w // CH
    n_main = (N // CH) * CH
    n_rem = N - n_main
    n_rows = SLAB * n_ch
    n_c16 = b_per_w // L

    uid2d = jnp.asarray(user_ids, jnp.int32).reshape(B // CH, CH)
    iid2d = jnp.asarray(item_ids, jnp.int32).reshape(B // CH, CH)
    gb16 = jnp.broadcast_to(jnp.asarray(global_bias, jnp.float32).reshape(1), (L,))
    uflats = _slab_flats(user_emb, n_main)
    iflats = _slab_flats(item_emb, n_main)
    rem_u = lax.slice(user_emb, (n_main, 0), (N, F))
    rem_i = lax.slice(item_emb, (n_main, 0), (N, F))

    mesh = plsc.VectorSubcoreMesh(core_axis_name="c", subcore_axis_name="s")

    def _stage_offsets(ids_ref, offs_ref):
        def offs_body(m, _):
            def sub_body(s, _):
                cols = pl.ds(s * L, L)
                r = ids_ref[m, cols]
                rc = jnp.minimum(r, n_main - 1)
                b0 = ((rc >> 7) << 10) + (rc & 127)
                for i1 in range(SLAB):
                    offs_ref[i1 * n_ch + m, cols] = b0 + i1 * CH
                return 0
            lax.fori_loop(0, CH // L, sub_body, 0)
            return 0
        lax.fori_loop(0, n_ch, offs_body, 0)

    def _fire_gathers(flats, offs_ref, buf2, sem):
        copies = []
        for t in range(NSLAB):
            for j in range(n_rows):
                i1, m = j // n_ch, j % n_ch
                copies.append(pltpu.async_copy(
                    flats[t].at[offs_ref.at[j]],
                    buf2.at[SLAB * t + i1, pl.ds(m * CH, CH)], sem))
        return copies

    def _fixup(ids_ref, buf2, rem):
        def fix_body(c16, _):
            m = c16 // (CH // L)
            cols = pl.ds((c16 % (CH // L)) * L, L)
            r = ids_ref[m, cols]
            rmax = lax.reduce_max(r, axes=(0,))

            @pl.when(rmax >= n_main)
            def _():
                tail = r >= n_main
                rr = jnp.maximum(r - n_main, 0)
                bcols = pl.ds(c16 * L, L)
                for c in range(F):
                    cur = buf2[c, bcols]
                    g = plsc.load_gather(
                        rem, [rr, jnp.full((L,), c, jnp.int32)])
                    buf2[c, bcols] = jnp.where(tail, g, cur)
            return 0
        lax.fori_loop(0, n_c16, fix_body, 0)

    @functools.partial(
        pl.kernel,
        mesh=mesh,
        compiler_params=_params,
        out_type=jax.ShapeDtypeStruct((F, B), jnp.float32),
        scratch_types=[
            pltpu.VMEM((n_ch, CH), jnp.int32),
            pltpu.VMEM((n_rows, CH), jnp.int32),
            pltpu.VMEM((F, b_per_w), jnp.float32),
            pltpu.VMEM((n_rem, F), jnp.float32),
            pltpu.SemaphoreType.DMA,
        ],
    )
    def _k1(uid_hbm, uf0, uf1, uf2, uf3, remu_hbm,
            ug_hbm, uidx, uoffs, ubuf2, remu, sem):
        ufs = (uf0, uf1, uf2, uf3)
        wid = lax.axis_index("s") * NC + lax.axis_index("c")
        pltpu.sync_copy(uid_hbm.at[pl.ds(wid * n_ch, n_ch)], uidx)
        pltpu.sync_copy(remu_hbm, remu)
        _stage_offsets(uidx, uoffs)
        for c in _fire_gathers(ufs, uoffs, ubuf2, sem):
            c.wait()
        _fixup(uidx, ubuf2, remu)
        pltpu.sync_copy(ubuf2, ug_hbm.at[:, pl.ds(wid * b_per_w, b_per_w)])

    @functools.partial(
        pl.kernel,
        mesh=mesh,
        compiler_params=_params,
        out_type=jax.ShapeDtypeStruct((B,), jnp.float32),
        scratch_types=[
            pltpu.VMEM((n_ch, CH), jnp.int32),
            pltpu.VMEM((n_ch, CH), jnp.int32),
            pltpu.VMEM((n_rows, CH), jnp.int32),
            pltpu.VMEM((F, b_per_w), jnp.float32),
            pltpu.VMEM((F, b_per_w), jnp.float32),
            pltpu.VMEM((n_ch, CH), jnp.float32),
            pltpu.VMEM((n_ch, CH), jnp.float32),
            pltpu.VMEM((n_rem, F), jnp.float32),
            pltpu.VMEM((b_per_w,), jnp.float32),
            pltpu.VMEM((L,), jnp.float32),
            pltpu.SemaphoreType.DMA,
        ],
    )
    def _k2(uid_hbm, iid_hbm, if0, if1, if2, if3, remi_hbm,
            ubias_hbm, ibias_hbm, gb_hbm, ug_hbm,
            out_hbm,
            uidx, iidx, ioffs, vbuf2, uv, bu, bi, remi, outb, gbv, sem):
        ifs = (if0, if1, if2, if3)
        wid = lax.axis_index("s") * NC + lax.axis_index("c")
        base = wid * b_per_w
        pltpu.sync_copy(uid_hbm.at[pl.ds(wid * n_ch, n_ch)], uidx)
        pltpu.sync_copy(iid_hbm.at[pl.ds(wid * n_ch, n_ch)], iidx)
        pltpu.sync_copy(gb_hbm, gbv)
        pltpu.sync_copy(remi_hbm, remi)
        _stage_offsets(iidx, ioffs)
        copies = _fire_gathers(ifs, ioffs, vbuf2, sem)
        for m in range(n_ch):
            copies.append(pltpu.async_copy(
                ubias_hbm.at[uidx.at[m]], bu.at[m], sem))
            copies.append(pltpu.async_copy(
                ibias_hbm.at[iidx.at[m]], bi.at[m], sem))
        pltpu.sync_copy(ug_hbm.at[:, pl.ds(base, b_per_w)], uv)
        for c in copies:
            c.wait()
        _fixup(iidx, vbuf2, remi)

        gv = gbv[...]

        def acc_body(c16, _):
            m = c16 // (CH // L)
            cols = pl.ds((c16 % (CH // L)) * L, L)
            bcols = pl.ds(c16 * L, L)
            acc = gv + bu[m, cols] + bi[m, cols]
            for c in range(F):
                acc = acc + uv[c, bcols] * vbuf2[c, bcols]
            outb[bcols] = acc
            return 0

        lax.fori_loop(0, n_c16, acc_body, 0)
        pltpu.sync_copy(outb, out_hbm.at[pl.ds(base, b_per_w)])

    ug = _k1(uid2d, *uflats, rem_u)
    return _k2(uid2d, iid2d, *iflats, rem_i,
               user_bias, item_bias, gb16, ug)
